# baseline (speedup 1.0000x reference)
import functools

import jax
import jax.numpy as jnp
from jax import lax
from jax.experimental import pallas as pl
from jax.experimental.pallas import tpu as pltpu
from jax.experimental.pallas import tpu_sc as plsc

N_NODES = 100000
N_EDGES = 3200000
D_EDGE = 16
D_NODE = 128
D_GLOBAL = 32

_BT = 5120


def _prep_body(ea_ref, *col_refs):
    xt = ea_ref[...].T
    for t in range(D_EDGE):
        col_refs[t][...] = xt[t, :]


def _prep(edge_attributes):
    nb = N_EDGES // _BT
    return pl.pallas_call(
        _prep_body,
        grid=(nb,),
        in_specs=[pl.BlockSpec((_BT, D_EDGE), lambda i: (i, 0))],
        out_specs=[pl.BlockSpec((_BT,), lambda i: (i,))] * D_EDGE,
        out_shape=[jax.ShapeDtypeStruct((N_EDGES,), jnp.float32)] * D_EDGE,
    )(edge_attributes)


_CK = 3200
_NCH = N_EDGES // _CK
_G = _CK // 16


def _seg2_body(dst_hbm, src_hbm, *rest):
    col_hbms = rest[:D_EDGE]
    rect_hbm, sent_hbm = rest[D_EDGE], rest[D_EDGE + 1]
    (idx_va, idx_vb, col_va, col_vb, acc, semi, semc) = rest[D_EDGE + 2:]

    c = lax.axis_index("c")
    s = lax.axis_index("s")

    z = jnp.zeros((16,), jnp.float32)

    @pl.loop(0, N_NODES // 16)
    def _(i):
        acc[pl.ds(i * 16, 16)] = z

    def issue(k, idx_v, col_v):
        @pl.when(c == 0)
        def _():
            pltpu.async_copy(dst_hbm.at[pl.ds(k * _CK, _CK)], idx_v, semi)

        @pl.when(c == 1)
        def _():
            pltpu.async_copy(src_hbm.at[pl.ds(k * _CK, _CK)], idx_v, semi)

        for t in range(D_EDGE):
            @pl.when(s == t)
            def _():
                pltpu.async_copy(col_hbms[t].at[pl.ds(k * _CK, _CK)], col_v, semc)

    def wait(idx_v, col_v):
        pltpu.make_async_copy(dst_hbm.at[pl.ds(0, _CK)], idx_v, semi).wait()
        pltpu.make_async_copy(col_hbms[0].at[pl.ds(0, _CK)], col_v, semc).wait()

    def compute(idx_v, col_v):
        @plsc.parallel_loop(0, _G, unroll=8)
        def _(g):
            idx = idx_v[pl.ds(g * 16, 16)]
            v = col_v[pl.ds(g * 16, 16)]
            plsc.addupdate_scatter(acc, [idx], v)

    issue(0, idx_va, col_va)
    issue(1, idx_vb, col_vb)

    @pl.loop(0, _NCH, step=2)
    def _(k):
        for (kk, idx_v, col_v) in ((k, idx_va, col_va), (k + 1, idx_vb, col_vb)):
            wait(idx_v, col_v)
            compute(idx_v, col_v)

            @pl.when(kk + 2 < _NCH)
            def _():
                issue(kk + 2, idx_v, col_v)

    @pl.when(c == 0)
    def _():
        pltpu.sync_copy(acc, rect_hbm.at[s])

    @pl.when(c == 1)
    def _():
        pltpu.sync_copy(acc, sent_hbm.at[s])


_seg2 = functools.partial(
    pl.kernel,
    out_type=[jax.ShapeDtypeStruct((D_EDGE, N_NODES), jnp.float32),
              jax.ShapeDtypeStruct((D_EDGE, N_NODES), jnp.float32)],
    mesh=plsc.VectorSubcoreMesh(core_axis_name="c", subcore_axis_name="s"),
    scratch_types=[
        pltpu.VMEM((_CK,), jnp.int32),
        pltpu.VMEM((_CK,), jnp.int32),
        pltpu.VMEM((_CK,), jnp.float32),
        pltpu.VMEM((_CK,), jnp.float32),
        pltpu.VMEM((N_NODES,), jnp.float32),
        pltpu.SemaphoreType.DMA,
        pltpu.SemaphoreType.DMA,
    ],
    compiler_params=pltpu.CompilerParams(use_tc_tiling_on_sc=False,
                                         needs_layout_passes=False),
)(_seg2_body)


_BM = 2048


def _mm_body(rect_ref, sent_ref, node_ref, g_ref, w_ref, b_ref, out_ref):
    acc = jnp.dot(node_ref[...], w_ref[2 * D_EDGE:2 * D_EDGE + D_NODE, :],
                  preferred_element_type=jnp.float32)
    acc += lax.dot_general(rect_ref[...], w_ref[:D_EDGE, :],
                           (((0,), (0,)), ((), ())),
                           preferred_element_type=jnp.float32)
    acc += lax.dot_general(sent_ref[...], w_ref[D_EDGE:2 * D_EDGE, :],
                           (((0,), (0,)), ((), ())),
                           preferred_element_type=jnp.float32)
    acc += jnp.dot(g_ref[...], w_ref[2 * D_EDGE + D_NODE:, :],
                   preferred_element_type=jnp.float32)
    out_ref[...] = acc + b_ref[...]


def _matmul(rect, sent, node, g2, w, b2):
    d_in = 2 * D_EDGE + D_NODE + D_GLOBAL
    return pl.pallas_call(
        _mm_body,
        grid=(pl.cdiv(N_NODES, _BM),),
        in_specs=[
            pl.BlockSpec((D_EDGE, _BM), lambda i: (0, i)),
            pl.BlockSpec((D_EDGE, _BM), lambda i: (0, i)),
            pl.BlockSpec((_BM, D_NODE), lambda i: (i, 0)),
            pl.BlockSpec((1, D_GLOBAL), lambda i: (0, 0)),
            pl.BlockSpec((d_in, D_NODE), lambda i: (0, 0)),
            pl.BlockSpec((1, D_NODE), lambda i: (0, 0)),
        ],
        out_specs=pl.BlockSpec((_BM, D_NODE), lambda i: (i, 0)),
        out_shape=jax.ShapeDtypeStruct((N_NODES, D_NODE), jnp.float32),
    )(rect, sent, node, g2, w, b2)


def kernel(node_attributes, edge_attributes, global_attributes, edge_index, W, b):
    dst1 = edge_index[1]
    src1 = edge_index[0]
    cols = _prep(edge_attributes)
    rect, sent = _seg2(dst1, src1, *cols)
    return _matmul(rect, sent, node_attributes,
                   global_attributes.reshape(1, D_GLOBAL), W,
                   b.reshape(1, D_NODE))


# speedup vs baseline: 1.1423x; 1.1423x over previous
import functools

import jax
import jax.numpy as jnp
from jax import lax
from jax.experimental import pallas as pl
from jax.experimental.pallas import tpu as pltpu
from jax.experimental.pallas import tpu_sc as plsc

N_NODES = 100000
N_EDGES = 3200000
D_EDGE = 16
D_NODE = 128
D_GLOBAL = 32

_BT = 25600


def _prep_body(ea_ref, *col_refs):
    eye = jnp.eye(D_EDGE, dtype=jnp.float32)
    xt = lax.dot_general(eye, ea_ref[...], (((1,), (1,)), ((), ())),
                         preferred_element_type=jnp.float32)
    for t in range(D_EDGE):
        col_refs[t][...] = xt[t, :]


def _prep(edge_attributes):
    nb = N_EDGES // _BT
    return pl.pallas_call(
        _prep_body,
        grid=(nb,),
        in_specs=[pl.BlockSpec((_BT, D_EDGE), lambda i: (i, 0))],
        out_specs=[pl.BlockSpec((_BT,), lambda i: (i,))] * D_EDGE,
        out_shape=[jax.ShapeDtypeStruct((N_EDGES,), jnp.float32)] * D_EDGE,
    )(edge_attributes)


_CK = 3200
_NCH = N_EDGES // _CK
_G = _CK // 16


def _seg2_body(dst_hbm, src_hbm, *rest):
    col_hbms = rest[:D_EDGE]
    rect_hbm, sent_hbm = rest[D_EDGE], rest[D_EDGE + 1]
    (idx_va, idx_vb, col_va, col_vb, acc, semi, semc) = rest[D_EDGE + 2:]

    c = lax.axis_index("c")
    s = lax.axis_index("s")

    z = jnp.zeros((16,), jnp.float32)

    @pl.loop(0, N_NODES // 16)
    def _(i):
        acc[pl.ds(i * 16, 16)] = z

    def issue(k, idx_v, col_v):
        @pl.when(c == 0)
        def _():
            pltpu.async_copy(dst_hbm.at[pl.ds(k * _CK, _CK)], idx_v, semi)

        @pl.when(c == 1)
        def _():
            pltpu.async_copy(src_hbm.at[pl.ds(k * _CK, _CK)], idx_v, semi)

        for t in range(D_EDGE):
            @pl.when(s == t)
            def _():
                pltpu.async_copy(col_hbms[t].at[pl.ds(k * _CK, _CK)], col_v, semc)

    def wait(idx_v, col_v):
        pltpu.make_async_copy(dst_hbm.at[pl.ds(0, _CK)], idx_v, semi).wait()
        pltpu.make_async_copy(col_hbms[0].at[pl.ds(0, _CK)], col_v, semc).wait()

    def compute(idx_v, col_v):
        @plsc.parallel_loop(0, _G, unroll=8)
        def _(g):
            idx = idx_v[pl.ds(g * 16, 16)]
            v = col_v[pl.ds(g * 16, 16)]
            plsc.addupdate_scatter(acc, [idx], v)

    issue(0, idx_va, col_va)
    issue(1, idx_vb, col_vb)

    @pl.loop(0, _NCH, step=2)
    def _(k):
        for (kk, idx_v, col_v) in ((k, idx_va, col_va), (k + 1, idx_vb, col_vb)):
            wait(idx_v, col_v)
            compute(idx_v, col_v)

            @pl.when(kk + 2 < _NCH)
            def _():
                issue(kk + 2, idx_v, col_v)

    @pl.when(c == 0)
    def _():
        pltpu.sync_copy(acc, rect_hbm.at[s])

    @pl.when(c == 1)
    def _():
        pltpu.sync_copy(acc, sent_hbm.at[s])


_seg2 = functools.partial(
    pl.kernel,
    out_type=[jax.ShapeDtypeStruct((D_EDGE, N_NODES), jnp.float32),
              jax.ShapeDtypeStruct((D_EDGE, N_NODES), jnp.float32)],
    mesh=plsc.VectorSubcoreMesh(core_axis_name="c", subcore_axis_name="s"),
    scratch_types=[
        pltpu.VMEM((_CK,), jnp.int32),
        pltpu.VMEM((_CK,), jnp.int32),
        pltpu.VMEM((_CK,), jnp.float32),
        pltpu.VMEM((_CK,), jnp.float32),
        pltpu.VMEM((N_NODES,), jnp.float32),
        pltpu.SemaphoreType.DMA,
        pltpu.SemaphoreType.DMA,
    ],
    compiler_params=pltpu.CompilerParams(use_tc_tiling_on_sc=False,
                                         needs_layout_passes=False),
)(_seg2_body)


_BM = 2048


def _mm_body(rect_ref, sent_ref, node_ref, g_ref, w_ref, b_ref, out_ref):
    acc = jnp.dot(node_ref[...], w_ref[2 * D_EDGE:2 * D_EDGE + D_NODE, :],
                  preferred_element_type=jnp.float32)
    acc += lax.dot_general(rect_ref[...], w_ref[:D_EDGE, :],
                           (((0,), (0,)), ((), ())),
                           preferred_element_type=jnp.float32)
    acc += lax.dot_general(sent_ref[...], w_ref[D_EDGE:2 * D_EDGE, :],
                           (((0,), (0,)), ((), ())),
                           preferred_element_type=jnp.float32)
    acc += jnp.dot(g_ref[...], w_ref[2 * D_EDGE + D_NODE:, :],
                   preferred_element_type=jnp.float32)
    out_ref[...] = acc + b_ref[...]


def _matmul(rect, sent, node, g2, w, b2):
    d_in = 2 * D_EDGE + D_NODE + D_GLOBAL
    return pl.pallas_call(
        _mm_body,
        grid=(pl.cdiv(N_NODES, _BM),),
        in_specs=[
            pl.BlockSpec((D_EDGE, _BM), lambda i: (0, i)),
            pl.BlockSpec((D_EDGE, _BM), lambda i: (0, i)),
            pl.BlockSpec((_BM, D_NODE), lambda i: (i, 0)),
            pl.BlockSpec((1, D_GLOBAL), lambda i: (0, 0)),
            pl.BlockSpec((d_in, D_NODE), lambda i: (0, 0)),
            pl.BlockSpec((1, D_NODE), lambda i: (0, 0)),
        ],
        out_specs=pl.BlockSpec((_BM, D_NODE), lambda i: (i, 0)),
        out_shape=jax.ShapeDtypeStruct((N_NODES, D_NODE), jnp.float32),
    )(rect, sent, node, g2, w, b2)


def kernel(node_attributes, edge_attributes, global_attributes, edge_index, W, b):
    dst1 = edge_index[1]
    src1 = edge_index[0]
    cols = _prep(edge_attributes)
    rect, sent = _seg2(dst1, src1, *cols)
    return _matmul(rect, sent, node_attributes,
                   global_attributes.reshape(1, D_GLOBAL), W,
                   b.reshape(1, D_NODE))


# speedup vs baseline: 1.2173x; 1.0656x over previous
import functools

import jax
import jax.numpy as jnp
from jax import lax
from jax.experimental import pallas as pl
from jax.experimental.pallas import tpu as pltpu
from jax.experimental.pallas import tpu_sc as plsc

N_NODES = 100000
N_EDGES = 3200000
D_EDGE = 16
D_NODE = 128
D_GLOBAL = 32

_BT = 25600


def _prep_body(ea_ref, ei_ref, dst_ref, src_ref, *col_refs):
    dst_ref[...] = ei_ref[1, :]
    src_ref[...] = ei_ref[0, :]
    eye = jnp.eye(D_EDGE, dtype=jnp.float32)
    xt = lax.dot_general(eye, ea_ref[...], (((1,), (1,)), ((), ())),
                         preferred_element_type=jnp.float32)
    for t in range(D_EDGE):
        col_refs[t][...] = xt[t, :]


def _prep(edge_attributes, edge_index):
    nb = N_EDGES // _BT
    out = pl.pallas_call(
        _prep_body,
        grid=(nb,),
        in_specs=[pl.BlockSpec((_BT, D_EDGE), lambda i: (i, 0)),
                  pl.BlockSpec((2, _BT), lambda i: (0, i))],
        out_specs=[pl.BlockSpec((_BT,), lambda i: (i,))] * (D_EDGE + 2),
        out_shape=[jax.ShapeDtypeStruct((N_EDGES,), jnp.int32)] * 2
        + [jax.ShapeDtypeStruct((N_EDGES,), jnp.float32)] * D_EDGE,
    )(edge_attributes, edge_index)
    return out[0], out[1], out[2:]


_CK = 3200
_NCH = N_EDGES // _CK
_G = _CK // 16


def _seg2_body(dst_hbm, src_hbm, *rest):
    col_hbms = rest[:D_EDGE]
    rect_hbm, sent_hbm = rest[D_EDGE], rest[D_EDGE + 1]
    (idx_va, idx_vb, col_va, col_vb, acc, semi, semc) = rest[D_EDGE + 2:]

    c = lax.axis_index("c")
    s = lax.axis_index("s")

    z = jnp.zeros((16,), jnp.float32)

    @pl.loop(0, N_NODES // 16)
    def _(i):
        acc[pl.ds(i * 16, 16)] = z

    def issue(k, idx_v, col_v):
        @pl.when(c == 0)
        def _():
            pltpu.async_copy(dst_hbm.at[pl.ds(k * _CK, _CK)], idx_v, semi)

        @pl.when(c == 1)
        def _():
            pltpu.async_copy(src_hbm.at[pl.ds(k * _CK, _CK)], idx_v, semi)

        for t in range(D_EDGE):
            @pl.when(s == t)
            def _():
                pltpu.async_copy(col_hbms[t].at[pl.ds(k * _CK, _CK)], col_v, semc)

    def wait(idx_v, col_v):
        pltpu.make_async_copy(dst_hbm.at[pl.ds(0, _CK)], idx_v, semi).wait()
        pltpu.make_async_copy(col_hbms[0].at[pl.ds(0, _CK)], col_v, semc).wait()

    def compute(idx_v, col_v):
        @plsc.parallel_loop(0, _G, unroll=8)
        def _(g):
            idx = idx_v[pl.ds(g * 16, 16)]
            v = col_v[pl.ds(g * 16, 16)]
            plsc.addupdate_scatter(acc, [idx], v)

    issue(0, idx_va, col_va)
    issue(1, idx_vb, col_vb)

    @pl.loop(0, _NCH, step=2)
    def _(k):
        for (kk, idx_v, col_v) in ((k, idx_va, col_va), (k + 1, idx_vb, col_vb)):
            wait(idx_v, col_v)
            compute(idx_v, col_v)

            @pl.when(kk + 2 < _NCH)
            def _():
                issue(kk + 2, idx_v, col_v)

    @pl.when(c == 0)
    def _():
        pltpu.sync_copy(acc, rect_hbm.at[s])

    @pl.when(c == 1)
    def _():
        pltpu.sync_copy(acc, sent_hbm.at[s])


_seg2 = functools.partial(
    pl.kernel,
    out_type=[jax.ShapeDtypeStruct((D_EDGE, N_NODES), jnp.float32),
              jax.ShapeDtypeStruct((D_EDGE, N_NODES), jnp.float32)],
    mesh=plsc.VectorSubcoreMesh(core_axis_name="c", subcore_axis_name="s"),
    scratch_types=[
        pltpu.VMEM((_CK,), jnp.int32),
        pltpu.VMEM((_CK,), jnp.int32),
        pltpu.VMEM((_CK,), jnp.float32),
        pltpu.VMEM((_CK,), jnp.float32),
        pltpu.VMEM((N_NODES,), jnp.float32),
        pltpu.SemaphoreType.DMA,
        pltpu.SemaphoreType.DMA,
    ],
    compiler_params=pltpu.CompilerParams(use_tc_tiling_on_sc=False,
                                         needs_layout_passes=False),
)(_seg2_body)


_BM = 2048


def _mm_body(rect_ref, sent_ref, node_ref, g_ref, w_ref, b_ref, out_ref):
    acc = jnp.dot(node_ref[...], w_ref[2 * D_EDGE:2 * D_EDGE + D_NODE, :],
                  preferred_element_type=jnp.float32)
    acc += lax.dot_general(rect_ref[...], w_ref[:D_EDGE, :],
                           (((0,), (0,)), ((), ())),
                           preferred_element_type=jnp.float32)
    acc += lax.dot_general(sent_ref[...], w_ref[D_EDGE:2 * D_EDGE, :],
                           (((0,), (0,)), ((), ())),
                           preferred_element_type=jnp.float32)
    acc += jnp.dot(g_ref[...], w_ref[2 * D_EDGE + D_NODE:, :],
                   preferred_element_type=jnp.float32)
    out_ref[...] = acc + b_ref[...]


def _matmul(rect, sent, node, g2, w, b2):
    d_in = 2 * D_EDGE + D_NODE + D_GLOBAL
    return pl.pallas_call(
        _mm_body,
        grid=(pl.cdiv(N_NODES, _BM),),
        in_specs=[
            pl.BlockSpec((D_EDGE, _BM), lambda i: (0, i)),
            pl.BlockSpec((D_EDGE, _BM), lambda i: (0, i)),
            pl.BlockSpec((_BM, D_NODE), lambda i: (i, 0)),
            pl.BlockSpec((1, D_GLOBAL), lambda i: (0, 0)),
            pl.BlockSpec((d_in, D_NODE), lambda i: (0, 0)),
            pl.BlockSpec((1, D_NODE), lambda i: (0, 0)),
        ],
        out_specs=pl.BlockSpec((_BM, D_NODE), lambda i: (i, 0)),
        out_shape=jax.ShapeDtypeStruct((N_NODES, D_NODE), jnp.float32),
    )(rect, sent, node, g2, w, b2)


def kernel(node_attributes, edge_attributes, global_attributes, edge_index, W, b):
    dst1, src1, cols = _prep(edge_attributes, edge_index)
    rect, sent = _seg2(dst1, src1, *cols)
    return _matmul(rect, sent, node_attributes,
                   global_attributes.reshape(1, D_GLOBAL), W,
                   b.reshape(1, D_NODE))
